# baseline (speedup 1.0000x reference)
import functools

import jax
import jax.numpy as jnp
from jax import lax
from jax.experimental import pallas as pl
from jax.experimental.pallas import tpu as pltpu
from jax.experimental.pallas import tpu_sc as plsc

VOCAB = 1000000
DIM = 64
B = 1024
L = 200
ALPHA = 5.0
SCALE = ALPHA / (L ** 0.5)

N_ROWS = B * L
N_ELEMS = N_ROWS * DIM
FLAT_COLS = 128
FLAT_ROWS = N_ELEMS // FLAT_COLS

NC, NS = 2, 16
NW = NC * NS
RPW = N_ROWS // NW
CH = 128
NCHUNK = RPW // CH
NBUF = 2
CHF = CH * DIM


def _sc_body(ids_hbm, table_hbm, noise_hbm, out_hbm,
             idx_v, idxhi_v, rows_v, sel_v, noise_v,
             sem_g, sem_n, sem_w):
    wid = lax.axis_index("s") * NC + lax.axis_index("c")
    base = wid * RPW
    pltpu.sync_copy(ids_hbm.at[pl.ds(base, RPW)], idx_v)

    @pl.loop(0, RPW // 16, step=1, unroll=8)
    def _shift(k):
        o = pl.multiple_of(k * 16, 16)
        idxhi_v[pl.ds(o, 16)] = jax.lax.shift_right_logical(
            idx_v[pl.ds(o, 16)], 1)

    def start_gather(c, b):
        off = pl.multiple_of(c * CH, 8)
        pltpu.async_copy(
            table_hbm.at[idxhi_v.at[pl.ds(off, CH)]], rows_v.at[b],
            sem_g.at[b])

    def wait_gather(b):
        pltpu.make_async_copy(
            table_hbm.at[pl.ds(0, CH)], rows_v.at[b], sem_g.at[b]).wait()

    def start_noise(c, b):
        off = pl.multiple_of((base + c * CH) * DIM, CHF)
        pltpu.async_copy(
            noise_hbm.at[pl.ds(off, CHF)], noise_v.at[b], sem_n.at[b])

    def wait_noise(b):
        pltpu.make_async_copy(
            noise_hbm.at[pl.ds(0, CHF)], noise_v.at[b], sem_n.at[b]).wait()

    def select_add(c, b):
        off = pl.multiple_of(c * CH, 8)
        lanes = lax.iota(jnp.int32, 16)

        @pl.loop(0, CH // 16, step=1)
        def _grp(q):
            t0 = q * 16
            idx16 = idx_v[pl.ds(off + t0, 16)]
            col0 = lax.rem(idx16, 2) * DIM
            row16 = t0 + lanes
            dst0 = (t0 + lanes) * DIM
            for d in range(DIM):
                v = plsc.load_gather(rows_v.at[b], [row16, col0 + d])
                nz = plsc.load_gather(noise_v.at[b], [dst0 + d])
                plsc.store_scatter(sel_v.at[b], [dst0 + d], v + nz)

    def start_write(c, b):
        off = pl.multiple_of((base + c * CH) * DIM, CHF)
        pltpu.async_copy(
            sel_v.at[b], out_hbm.at[pl.ds(off, CHF)], sem_w.at[b])

    def wait_write(b):
        pltpu.make_async_copy(
            sel_v.at[b], out_hbm.at[pl.ds(0, CHF)], sem_w.at[b]).wait()

    for b in range(NBUF):
        start_gather(b, b)
        start_noise(b, b)

    @pl.loop(0, NCHUNK - NBUF, step=NBUF)
    def _steady(g):
        for b in range(NBUF):
            c = g + b
            wait_gather(b)
            wait_noise(b)
            select_add(c, b)
            start_write(c, b)
            wait_write(b)
            start_gather(c + NBUF, b)
            start_noise(c + NBUF, b)

    for b in range(NBUF):
        c = NCHUNK - NBUF + b
        wait_gather(b)
        wait_noise(b)
        select_add(c, b)
        start_write(c, b)
        wait_write(b)


@functools.lru_cache(maxsize=None)
def _sc_gather():
    mesh = plsc.VectorSubcoreMesh(
        core_axis_name="c", subcore_axis_name="s",
        num_cores=NC, num_subcores=NS)
    return pl.kernel(
        _sc_body,
        mesh=mesh,
        out_type=jax.ShapeDtypeStruct((N_ELEMS,), jnp.float32),
        scratch_types=[
            pltpu.VMEM((RPW,), jnp.int32),
            pltpu.VMEM((RPW,), jnp.int32),
            pltpu.VMEM((NBUF, CH, FLAT_COLS), jnp.float32),
            pltpu.VMEM((NBUF, CHF), jnp.float32),
            pltpu.VMEM((NBUF, CHF), jnp.float32),
            pltpu.SemaphoreType.DMA((NBUF,)),
            pltpu.SemaphoreType.DMA((NBUF,)),
            pltpu.SemaphoreType.DMA((NBUF,)),
        ],
        compiler_params=pltpu.CompilerParams(
            use_tc_tiling_on_sc=False, needs_layout_passes=False),
    )


NR_BLK = 2048
N_BLKS = FLAT_ROWS // NR_BLK

_ROT = ((13, 15, 26, 6), (17, 29, 16, 24))
_KS = (0, 42, 0 ^ 42 ^ 0x1BD11BDA)


def _noise_body(out_ref):
    pid = pl.program_id(0)
    flat0 = (pid * (NR_BLK * FLAT_COLS)).astype(jnp.uint32)
    r = lax.broadcasted_iota(jnp.uint32, (NR_BLK, FLAT_COLS), 0)
    c = lax.broadcasted_iota(jnp.uint32, (NR_BLK, FLAT_COLS), 1)
    i = flat0 + r * jnp.uint32(FLAT_COLS) + c
    x0 = jnp.zeros_like(i)
    x1 = i + jnp.uint32(_KS[1])
    for g in range(5):
        for rot in _ROT[g % 2]:
            x0 = x0 + x1
            x1 = (x1 << jnp.uint32(rot)) | (x1 >> jnp.uint32(32 - rot))
            x1 = x1 ^ x0
        x0 = x0 + jnp.uint32(_KS[(g + 1) % 3])
        x1 = x1 + jnp.uint32(_KS[(g + 2) % 3] + g + 1)
    bits = x0 ^ x1
    fb = (bits >> jnp.uint32(9)) | jnp.uint32(0x3F800000)
    f = lax.bitcast_convert_type(fb, jnp.float32) - jnp.float32(1.0)
    out_ref[...] = f * jnp.float32(2.0 * SCALE) + jnp.float32(-SCALE)


_noise = pl.pallas_call(
    _noise_body,
    grid=(N_BLKS,),
    out_specs=pl.BlockSpec((NR_BLK, FLAT_COLS), lambda g: (g, 0)),
    out_shape=jax.ShapeDtypeStruct((FLAT_ROWS, FLAT_COLS), jnp.float32),
)


@jax.jit
def kernel(input_ids, table):
    ids = input_ids.reshape(-1)
    table2 = table.reshape(VOCAB // 2, 2 * DIM)
    noise = _noise().reshape(-1)
    out = _sc_gather()(ids, table2, noise)
    return out.reshape(B, L, DIM)


# speedup vs baseline: 1.6980x; 1.6980x over previous
import functools

import jax
import jax.numpy as jnp
from jax import lax
from jax.experimental import pallas as pl
from jax.experimental.pallas import tpu as pltpu
from jax.experimental.pallas import tpu_sc as plsc

VOCAB = 1000000
DIM = 64
B = 1024
L = 200
ALPHA = 5.0
SCALE = ALPHA / (L ** 0.5)

N_ROWS = B * L
N_ELEMS = N_ROWS * DIM
FLAT_COLS = 128
FLAT_ROWS = N_ELEMS // FLAT_COLS

NC, NS = 2, 16
NW = NC * NS
RPW = N_ROWS // NW
CH = 128
NCHUNK = RPW // CH
NBUF = 2
CHF = CH * DIM


def _sc_body(ids_hbm, table_hbm, noise_hbm, out_hbm,
             idx_v, rows_v, noise_v, sem_g, sem_n, sem_w):
    wid = lax.axis_index("s") * NC + lax.axis_index("c")
    base = wid * RPW
    pltpu.sync_copy(ids_hbm.at[pl.ds(base, RPW)], idx_v)

    def start_gather(c, b):
        off = pl.multiple_of(c * CH, 8)
        pltpu.async_copy(
            table_hbm.at[idx_v.at[pl.ds(off, CH)]], rows_v.at[b],
            sem_g.at[b])

    def wait_gather(b):
        pltpu.make_async_copy(
            table_hbm.at[pl.ds(0, CH)], rows_v.at[b], sem_g.at[b]).wait()

    def start_noise(c, b):
        off = pl.multiple_of((base + c * CH) * DIM, CHF)
        pltpu.async_copy(
            noise_hbm.at[pl.ds(off, CHF)], noise_v.at[b], sem_n.at[b])

    def wait_noise(b):
        pltpu.make_async_copy(
            noise_hbm.at[pl.ds(0, CHF)], noise_v.at[b], sem_n.at[b]).wait()

    def add_noise(c, b):
        @pl.loop(0, CH, step=1, unroll=4)
        def _tok(t):
            for j in range(DIM // 16):
                o = pl.multiple_of(t * DIM + j * 16, 16)
                rows_v[b, t, pl.ds(j * 16, 16)] = (
                    rows_v[b, t, pl.ds(j * 16, 16)]
                    + noise_v[b, pl.ds(o, 16)])

    def start_write(c, b):
        off = pl.multiple_of(base + c * CH, 8)
        pltpu.async_copy(
            rows_v.at[b], out_hbm.at[pl.ds(off, CH)], sem_w.at[b])

    def wait_write(b):
        pltpu.make_async_copy(
            rows_v.at[b], out_hbm.at[pl.ds(0, CH)], sem_w.at[b]).wait()

    for b in range(NBUF):
        start_gather(b, b)
        start_noise(b, b)

    @pl.loop(0, NCHUNK - NBUF, step=NBUF)
    def _steady(g):
        for b in range(NBUF):
            c = g + b
            wait_gather(b)
            wait_noise(b)
            add_noise(c, b)
            start_write(c, b)
            wait_write(b)
            start_gather(c + NBUF, b)
            start_noise(c + NBUF, b)

    for b in range(NBUF):
        c = NCHUNK - NBUF + b
        wait_gather(b)
        wait_noise(b)
        add_noise(c, b)
        start_write(c, b)
        wait_write(b)


@functools.lru_cache(maxsize=None)
def _sc_gather():
    mesh = plsc.VectorSubcoreMesh(
        core_axis_name="c", subcore_axis_name="s",
        num_cores=NC, num_subcores=NS)
    return pl.kernel(
        _sc_body,
        mesh=mesh,
        out_type=jax.ShapeDtypeStruct((N_ROWS, DIM), jnp.float32),
        scratch_types=[
            pltpu.VMEM((RPW,), jnp.int32),
            pltpu.VMEM((NBUF, CH, DIM), jnp.float32),
            pltpu.VMEM((NBUF, CHF), jnp.float32),
            pltpu.SemaphoreType.DMA((NBUF,)),
            pltpu.SemaphoreType.DMA((NBUF,)),
            pltpu.SemaphoreType.DMA((NBUF,)),
        ],
        compiler_params=pltpu.CompilerParams(
            use_tc_tiling_on_sc=False, needs_layout_passes=False),
    )


NR_BLK = 2048
N_BLKS = FLAT_ROWS // NR_BLK

_ROT = ((13, 15, 26, 6), (17, 29, 16, 24))
_KS = (0, 42, 0 ^ 42 ^ 0x1BD11BDA)


def _noise_body(out_ref):
    pid = pl.program_id(0)
    flat0 = (pid * (NR_BLK * FLAT_COLS)).astype(jnp.uint32)
    r = lax.broadcasted_iota(jnp.uint32, (NR_BLK, FLAT_COLS), 0)
    c = lax.broadcasted_iota(jnp.uint32, (NR_BLK, FLAT_COLS), 1)
    i = flat0 + r * jnp.uint32(FLAT_COLS) + c
    x0 = jnp.zeros_like(i)
    x1 = i + jnp.uint32(_KS[1])
    for g in range(5):
        for rot in _ROT[g % 2]:
            x0 = x0 + x1
            x1 = (x1 << jnp.uint32(rot)) | (x1 >> jnp.uint32(32 - rot))
            x1 = x1 ^ x0
        x0 = x0 + jnp.uint32(_KS[(g + 1) % 3])
        x1 = x1 + jnp.uint32(_KS[(g + 2) % 3] + g + 1)
    bits = x0 ^ x1
    fb = (bits >> jnp.uint32(9)) | jnp.uint32(0x3F800000)
    f = lax.bitcast_convert_type(fb, jnp.float32) - jnp.float32(1.0)
    out_ref[...] = f * jnp.float32(2.0 * SCALE) + jnp.float32(-SCALE)


_noise = pl.pallas_call(
    _noise_body,
    grid=(N_BLKS,),
    out_specs=pl.BlockSpec((NR_BLK, FLAT_COLS), lambda g: (g, 0)),
    out_shape=jax.ShapeDtypeStruct((FLAT_ROWS, FLAT_COLS), jnp.float32),
)


@jax.jit
def kernel(input_ids, table):
    ids = input_ids.reshape(-1)
    noise = _noise().reshape(-1)
    out = _sc_gather()(ids, table, noise)
    return out.reshape(B, L, DIM)


# speedup vs baseline: 1.8014x; 1.0609x over previous
import functools

import jax
import jax.numpy as jnp
from jax import lax
from jax.experimental import pallas as pl
from jax.experimental.pallas import tpu as pltpu
from jax.experimental.pallas import tpu_sc as plsc

VOCAB = 1000000
DIM = 64
B = 1024
L = 200
ALPHA = 5.0
SCALE = ALPHA / (L ** 0.5)

N_ROWS = B * L
N_ELEMS = N_ROWS * DIM
FLAT_COLS = 128
FLAT_ROWS = N_ELEMS // FLAT_COLS

NC, NS = 2, 16
NW = NC * NS
RPW = N_ROWS // NW
CH = 128
NCHUNK = RPW // CH
NBUF = 5
CHF = CH * DIM


def _sc_body(ids_hbm, table_hbm, noise_hbm, out_hbm,
             idx_v, rows_v, noise_v, sem_g, sem_n, sem_w):
    wid = lax.axis_index("s") * NC + lax.axis_index("c")
    base = wid * RPW
    pltpu.sync_copy(ids_hbm.at[pl.ds(base, RPW)], idx_v)

    def start_gather(c, b):
        off = pl.multiple_of(c * CH, 8)
        pltpu.async_copy(
            table_hbm.at[idx_v.at[pl.ds(off, CH)]], rows_v.at[b],
            sem_g.at[b])

    def wait_gather(b):
        pltpu.make_async_copy(
            table_hbm.at[pl.ds(0, CH)], rows_v.at[b], sem_g.at[b]).wait()

    def start_noise(c, b):
        off = pl.multiple_of((base + c * CH) * DIM, CHF)
        pltpu.async_copy(
            noise_hbm.at[pl.ds(off, CHF)], noise_v.at[b], sem_n.at[b])

    def wait_noise(b):
        pltpu.make_async_copy(
            noise_hbm.at[pl.ds(0, CHF)], noise_v.at[b], sem_n.at[b]).wait()

    def add_noise(c, b):
        @pl.loop(0, CH, step=1, unroll=4)
        def _tok(t):
            for j in range(DIM // 16):
                o = pl.multiple_of(t * DIM + j * 16, 16)
                rows_v[b, t, pl.ds(j * 16, 16)] = (
                    rows_v[b, t, pl.ds(j * 16, 16)]
                    + noise_v[b, pl.ds(o, 16)])

    def start_write(c, b):
        off = pl.multiple_of(base + c * CH, 8)
        pltpu.async_copy(
            rows_v.at[b], out_hbm.at[pl.ds(off, CH)], sem_w.at[b])

    def wait_write(b):
        pltpu.make_async_copy(
            rows_v.at[b], out_hbm.at[pl.ds(0, CH)], sem_w.at[b]).wait()

    for b in range(NBUF):
        start_gather(b, b)
        start_noise(b, b)

    @pl.loop(0, NCHUNK - NBUF, step=NBUF)
    def _steady(g):
        for b in range(NBUF):
            c = g + b
            wait_gather(b)
            wait_noise(b)
            add_noise(c, b)
            start_write(c, b)
            wait_write(b)
            start_gather(c + NBUF, b)
            start_noise(c + NBUF, b)

    for b in range(NBUF):
        c = NCHUNK - NBUF + b
        wait_gather(b)
        wait_noise(b)
        add_noise(c, b)
        start_write(c, b)
        wait_write(b)


@functools.lru_cache(maxsize=None)
def _sc_gather():
    mesh = plsc.VectorSubcoreMesh(
        core_axis_name="c", subcore_axis_name="s",
        num_cores=NC, num_subcores=NS)
    return pl.kernel(
        _sc_body,
        mesh=mesh,
        out_type=jax.ShapeDtypeStruct((N_ROWS, DIM), jnp.float32),
        scratch_types=[
            pltpu.VMEM((RPW,), jnp.int32),
            pltpu.VMEM((NBUF, CH, DIM), jnp.float32),
            pltpu.VMEM((NBUF, CHF), jnp.float32),
            pltpu.SemaphoreType.DMA((NBUF,)),
            pltpu.SemaphoreType.DMA((NBUF,)),
            pltpu.SemaphoreType.DMA((NBUF,)),
        ],
        compiler_params=pltpu.CompilerParams(
            use_tc_tiling_on_sc=False, needs_layout_passes=False),
    )


NR_BLK = 2048
N_BLKS = FLAT_ROWS // NR_BLK

_ROT = ((13, 15, 26, 6), (17, 29, 16, 24))
_KS = (0, 42, 0 ^ 42 ^ 0x1BD11BDA)


def _noise_body(out_ref):
    pid = pl.program_id(0)
    flat0 = (pid * (NR_BLK * FLAT_COLS)).astype(jnp.uint32)
    r = lax.broadcasted_iota(jnp.uint32, (NR_BLK, FLAT_COLS), 0)
    c = lax.broadcasted_iota(jnp.uint32, (NR_BLK, FLAT_COLS), 1)
    i = flat0 + r * jnp.uint32(FLAT_COLS) + c
    x0 = jnp.zeros_like(i)
    x1 = i + jnp.uint32(_KS[1])
    for g in range(5):
        for rot in _ROT[g % 2]:
            x0 = x0 + x1
            x1 = (x1 << jnp.uint32(rot)) | (x1 >> jnp.uint32(32 - rot))
            x1 = x1 ^ x0
        x0 = x0 + jnp.uint32(_KS[(g + 1) % 3])
        x1 = x1 + jnp.uint32(_KS[(g + 2) % 3] + g + 1)
    bits = x0 ^ x1
    fb = (bits >> jnp.uint32(9)) | jnp.uint32(0x3F800000)
    f = lax.bitcast_convert_type(fb, jnp.float32) - jnp.float32(1.0)
    out_ref[...] = f * jnp.float32(2.0 * SCALE) + jnp.float32(-SCALE)


_noise = pl.pallas_call(
    _noise_body,
    grid=(N_BLKS,),
    out_specs=pl.BlockSpec((NR_BLK, FLAT_COLS), lambda g: (g, 0)),
    out_shape=jax.ShapeDtypeStruct((FLAT_ROWS, FLAT_COLS), jnp.float32),
)


@jax.jit
def kernel(input_ids, table):
    ids = input_ids.reshape(-1)
    noise = _noise().reshape(-1)
    out = _sc_gather()(ids, table, noise)
    return out.reshape(B, L, DIM)
